# Initial kernel scaffold; baseline (speedup 1.0000x reference)
import jax
import jax.numpy as jnp
from jax.experimental import pallas as pl


def kernel(users, pos_items, neg_items, adj1_idx, adj1_val, adj2_idx, adj2_val, graph_idx, graph_val, embed_user, embed_item):
    raise NotImplementedError("write your pallas kernel here")


# speedup vs baseline: 1.0010x; 1.0010x over previous
import functools

import jax
import jax.numpy as jnp
from jax import lax
from jax.experimental import pallas as pl
from jax.experimental.pallas import tpu as pltpu
from jax.experimental.pallas import tpu_sc as plsc

_N_USERS = 20000
_N_ITEMS = 30000
_N = _N_USERS + _N_ITEMS
_EMB = 64
_NNZ = 1600000
_B = 4096
_N_LAYERS = 3
_CL_RATE = 0.2
_TEMP = 0.2
_DECAY = 1e-4

_INTERPRET = False


_CH = 512
_NCH = _B // _CH


def _rownorm(x):
    ss = jnp.sum(x * x, axis=1, keepdims=True)
    nrm = jnp.sqrt(ss)
    return x / jnp.maximum(nrm, 1e-12)


def _loss_body(users_col, users_row, pos_col, pos_row,
               ue, pe, ne, ue0, pe0, ne0,
               u1, u2, i1, i2,
               mf_ref, cl_ref, reg_ref,
               mcol_s, mrow_s, n1s, n2s):
    uen = _rownorm(ue[...])
    pen = _rownorm(pe[...])
    nen = _rownorm(ne[...])
    pos_s = jnp.sum(uen * pen, axis=1, keepdims=True)
    neg_s = jnp.sum(uen * nen, axis=1, keepdims=True)
    x = pos_s - neg_s
    sig = 1.0 / (1.0 + jnp.exp(-x))
    maxi = jnp.log(sig + 1e-6)
    mf_ref[0, 0] = -jnp.sum(maxi) / _B

    reg = 0.5 * (jnp.sum(ue0[...] * ue0[...]) + jnp.sum(pe0[...] * pe0[...])
                 + jnp.sum(ne0[...] * ne0[...]))
    reg_ref[0, 0] = _DECAY * (reg / _B)

    def one_cl(ids_col, ids_row, v1, v2):
        def mask_col_body(c, carry):
            rows = ids_col[pl.ds(c * _CH, _CH), :]
            eq = rows == ids_row[...]
            ii = lax.broadcasted_iota(jnp.int32, (_CH, _B), 0) + c * _CH
            jj = lax.broadcasted_iota(jnp.int32, (_CH, _B), 1)
            dup = jnp.any(eq & (jj < ii), axis=1, keepdims=True)
            mcol_s[pl.ds(c * _CH, _CH), :] = jnp.where(dup, 0.0, 1.0)
            return carry

        lax.fori_loop(0, _NCH, mask_col_body, 0)

        def mask_row_body(c, carry):
            cols = ids_row[:, pl.ds(c * _CH, _CH)]
            eq = ids_col[...] == cols
            kk = lax.broadcasted_iota(jnp.int32, (_B, _CH), 0)
            jj = lax.broadcasted_iota(jnp.int32, (_B, _CH), 1) + c * _CH
            dup = jnp.any(eq & (kk < jj), axis=0, keepdims=True)
            mrow_s[:, pl.ds(c * _CH, _CH)] = jnp.where(dup, 0.0, 1.0)
            return carry

        lax.fori_loop(0, _NCH, mask_row_body, 0)

        count = jnp.sum(mcol_s[...])
        n1s[...] = _rownorm(v1[...])
        n2s[...] = _rownorm(v2[...])

        def chunk_body(c, acc):
            n1c = n1s[pl.ds(c * _CH, _CH), :]
            n2c = n2s[pl.ds(c * _CH, _CH), :]
            posv = jnp.exp(jnp.sum(n1c * n2c, axis=1, keepdims=True) / _TEMP)
            s = lax.dot_general(n1c, n2s[...], (((1,), (1,)), ((), ())),
                                preferred_element_type=jnp.float32)
            e = jnp.exp(s / _TEMP) * mrow_s[...]
            ttl = jnp.sum(e, axis=1, keepdims=True)
            li = -jnp.log(posv / ttl + 1e-5)
            return acc + jnp.sum(li * mcol_s[pl.ds(c * _CH, _CH), :])

        tot = lax.fori_loop(0, _NCH, chunk_body, 0.0)
        return tot / count

    ucl = one_cl(users_col, users_row, u1, u2)
    icl = one_cl(pos_col, pos_row, i1, i2)
    cl_ref[0, 0] = _CL_RATE * (ucl + icl)


def _loss_call(users_col, users_row, pos_col, pos_row,
               ue, pe, ne, ue0, pe0, ne0, u1, u2, i1, i2):
    f32 = jnp.float32
    out = pl.pallas_call(
        _loss_body,
        out_shape=[jax.ShapeDtypeStruct((1, 1), f32)] * 3,
        out_specs=[pl.BlockSpec(memory_space=pltpu.SMEM)] * 3,
        scratch_shapes=[
            pltpu.VMEM((_B, 1), f32),
            pltpu.VMEM((1, _B), f32),
            pltpu.VMEM((_B, _EMB), f32),
            pltpu.VMEM((_B, _EMB), f32),
        ],
        interpret=_INTERPRET,
    )(users_col, users_row, pos_col, pos_row,
      ue, pe, ne, ue0, pe0, ne0, u1, u2, i1, i2)
    return out[0][0, 0], out[1][0, 0], out[2][0, 0]


def _spmm(idx, val, x):
    return jax.ops.segment_sum(val[:, None] * x[idx[1]], idx[0],
                               num_segments=x.shape[0])


def _propagate(all_emb, idx, val):
    acc = all_emb
    x = all_emb
    for _ in range(_N_LAYERS):
        x = _spmm(idx, val, x)
        acc = acc + x
    return acc * (1.0 / (_N_LAYERS + 1))


def kernel(users, pos_items, neg_items, adj1_idx, adj1_val, adj2_idx, adj2_val,
           graph_idx, graph_val, embed_user, embed_item):
    all_emb = jnp.concatenate([embed_user, embed_item], axis=0)

    lo_g = _propagate(all_emb, graph_idx, graph_val)
    lo_1 = _propagate(all_emb, adj1_idx, adj1_val)
    lo_2 = _propagate(all_emb, adj2_idx, adj2_val)

    users_i = users.astype(jnp.int32)
    pos_i = pos_items.astype(jnp.int32)
    neg_i = neg_items.astype(jnp.int32)
    pos_n = pos_i + _N_USERS

    ue = lo_g[users_i]
    pe = lo_g[pos_n]
    ne = lo_g[neg_i + _N_USERS]
    ue0 = embed_user[users_i]
    pe0 = embed_item[pos_i]
    ne0 = embed_item[neg_i]
    u1 = lo_1[users_i]
    u2 = lo_2[users_i]
    i1 = lo_1[pos_n]
    i2 = lo_2[pos_n]

    users_col = users_i.reshape(_B, 1)
    users_row = users_i.reshape(1, _B)
    pos_col = pos_i.reshape(_B, 1)
    pos_row = pos_i.reshape(1, _B)

    mf, cl, reg = _loss_call(users_col, users_row, pos_col, pos_row,
                             ue, pe, ne, ue0, pe0, ne0, u1, u2, i1, i2)
    return mf, cl, reg


# speedup vs baseline: 3.4680x; 3.4647x over previous
import functools

import jax
import jax.numpy as jnp
from jax import lax
from jax.experimental import pallas as pl
from jax.experimental.pallas import tpu as pltpu
from jax.experimental.pallas import tpu_sc as plsc

_N_USERS = 20000
_N_ITEMS = 30000
_N = _N_USERS + _N_ITEMS
_EMB = 64
_HALF = 32
_NNZ = 1600000
_B = 4096
_N_LAYERS = 3
_CL_RATE = 0.2
_TEMP = 0.2
_DECAY = 1e-4

_INTERPRET = False

_NC = 2
_NS = 16

_NNZ_PAD = 1638400
_EDGES_PER_TILE = _NNZ_PAD // _NS
_KB = 512
_SUB = 128
_NSUB = _KB // _SUB
_NBLK = _EDGES_PER_TILE // _KB
_IDX_ROWS = _NNZ_PAD // _SUB
_NPAD = 50048
_ROWS_PER_TILE = _NPAD // _NS

_mesh = plsc.VectorSubcoreMesh(core_axis_name="c", subcore_axis_name="s",
                               num_cores=_NC, num_subcores=_NS)


def _spmm_body(x_lo, x_hi, rows2, cols2, vals2, zeros, y_lo, y_hi,
               ridx2, cidx2, vbuf2, xbuf, acc, gsem, ssem):
    c = lax.axis_index("c")
    s = lax.axis_index("s")

    def half(xsrc, ydst):
        pltpu.sync_copy(zeros, acc.at[pl.ds(s * _ROWS_PER_TILE, _ROWS_PER_TILE)])
        plsc.subcore_barrier()

        def block(b, carry):
            rb = s * (_EDGES_PER_TILE // _SUB) + b * _NSUB
            pltpu.sync_copy(rows2.at[pl.ds(rb, _NSUB)], ridx2)
            pltpu.sync_copy(cols2.at[pl.ds(rb, _NSUB)], cidx2)
            pltpu.sync_copy(vals2.at[pl.ds(rb, _NSUB)], vbuf2)
            descs = [
                pltpu.async_copy(xsrc.at[cidx2.at[j]],
                                 xbuf.at[pl.ds(j * _SUB, _SUB)], gsem)
                for j in range(_NSUB)
            ]
            for d in descs:
                d.wait()
            for jj in range(_NSUB):
                def scale(g, carry2, jj=jj):
                    vv = vbuf2[jj, pl.ds(g * 16, 16)]
                    for l in range(16):
                        v = vv[l]
                        r = jj * _SUB + g * 16 + l
                        xbuf[r, pl.ds(0, 16)] = xbuf[r, pl.ds(0, 16)] * v
                        xbuf[r, pl.ds(16, 16)] = xbuf[r, pl.ds(16, 16)] * v
                    return carry2
                lax.fori_loop(0, _SUB // 16, scale, 0)
            descs = [
                pltpu.async_copy(xbuf.at[pl.ds(j * _SUB, _SUB)],
                                 acc.at[ridx2.at[j]], ssem, add=True)
                for j in range(_NSUB)
            ]
            for d in descs:
                d.wait()
            return carry

        lax.fori_loop(0, _NBLK, block, 0)
        plsc.subcore_barrier()
        pltpu.sync_copy(acc.at[pl.ds(s * _ROWS_PER_TILE, _ROWS_PER_TILE)],
                        ydst.at[pl.ds(s * _ROWS_PER_TILE, _ROWS_PER_TILE)])

    @pl.when(c == 0)
    def _():
        half(x_lo, y_lo)

    @pl.when(c == 1)
    def _():
        half(x_hi, y_hi)


_spmm_call = pl.kernel(
    _spmm_body,
    out_type=[jax.ShapeDtypeStruct((_NPAD, _HALF), jnp.float32)] * 2,
    mesh=_mesh,
    scratch_types=[
        pltpu.VMEM((_NSUB, _SUB), jnp.int32),
        pltpu.VMEM((_NSUB, _SUB), jnp.int32),
        pltpu.VMEM((_NSUB, _SUB), jnp.float32),
        pltpu.VMEM((_KB, _HALF), jnp.float32),
        pltpu.VMEM_SHARED((_NPAD, _HALF), jnp.float32),
        pltpu.SemaphoreType.DMA,
        pltpu.SemaphoreType.DMA,
    ],
    compiler_params=pltpu.CompilerParams(use_tc_tiling_on_sc=False),
)


_GROWS = _B // _NS
_GSUB = _GROWS // 128


def _gather_body(x0l, x0h, g1l, g1h, g2l, g2h, g3l, g3h,
                 a1l, a1h, a2l, a2h, a3l, a3h,
                 b1l, b1h, b2l, b2h, b3l, b3h,
                 uidx, pidx, nidx, gout,
                 ibuf, b0, b1, b2, b3, mbuf, gsem):
    c = lax.axis_index("c")
    s = lax.axis_index("s")

    def run(hh, tabs):
        x0t, g1t, g2t, g3t, a1t, a2t, a3t, b1t, b2t, b3t = tabs

        def g4(t0, t1, t2, t3):
            descs = []
            for j in range(_GSUB):
                dst = pl.ds(j * 128, 128)
                idxr = ibuf.at[s * _GSUB + j]
                descs += [
                    pltpu.async_copy(t0.at[idxr], b0.at[dst], gsem),
                    pltpu.async_copy(t1.at[idxr], b1.at[dst], gsem),
                    pltpu.async_copy(t2.at[idxr], b2.at[dst], gsem),
                    pltpu.async_copy(t3.at[idxr], b3.at[dst], gsem),
                ]
            for d in descs:
                d.wait()

        def g3only(t1, t2, t3):
            descs = []
            for j in range(_GSUB):
                dst = pl.ds(j * 128, 128)
                idxr = ibuf.at[s * _GSUB + j]
                descs += [
                    pltpu.async_copy(t1.at[idxr], b1.at[dst], gsem),
                    pltpu.async_copy(t2.at[idxr], b2.at[dst], gsem),
                    pltpu.async_copy(t3.at[idxr], b3.at[dst], gsem),
                ]
            for d in descs:
                d.wait()

        def mean4():
            def mrow(r, carry):
                for h2 in range(2):
                    sl = pl.ds(h2 * 16, 16)
                    mbuf[r, sl] = (b0[r, sl] + b1[r, sl] + b2[r, sl]
                                   + b3[r, sl]) * 0.25
                return carry
            lax.fori_loop(0, _GROWS, mrow, 0, unroll=4)

        def wout(q, src):
            pltpu.sync_copy(src, gout.at[hh, q, pl.ds(s * _GROWS, _GROWS)])

        for idx_hbm, q0, has_cl in [(uidx, 0, True), (pidx, 4, True),
                                    (nidx, 8, False)]:
            pltpu.sync_copy(idx_hbm, ibuf)
            g4(x0t, g1t, g2t, g3t)
            wout(q0, b0)
            mean4()
            wout(q0 + 1, mbuf)
            if has_cl:
                g3only(a1t, a2t, a3t)
                mean4()
                wout(q0 + 2, mbuf)
                g3only(b1t, b2t, b3t)
                mean4()
                wout(q0 + 3, mbuf)

    @pl.when(c == 0)
    def _():
        run(0, (x0l, g1l, g2l, g3l, a1l, a2l, a3l, b1l, b2l, b3l))

    @pl.when(c == 1)
    def _():
        run(1, (x0h, g1h, g2h, g3h, a1h, a2h, a3h, b1h, b2h, b3h))


_gather_call = pl.kernel(
    _gather_body,
    out_type=jax.ShapeDtypeStruct((2, 10, _B, _HALF), jnp.float32),
    mesh=_mesh,
    scratch_types=[
        pltpu.VMEM((_B // 128, 128), jnp.int32),
        pltpu.VMEM((_GROWS, _HALF), jnp.float32),
        pltpu.VMEM((_GROWS, _HALF), jnp.float32),
        pltpu.VMEM((_GROWS, _HALF), jnp.float32),
        pltpu.VMEM((_GROWS, _HALF), jnp.float32),
        pltpu.VMEM((_GROWS, _HALF), jnp.float32),
        pltpu.SemaphoreType.DMA,
    ],
    compiler_params=pltpu.CompilerParams(use_tc_tiling_on_sc=False),
)


_CH = 512
_NCH = _B // _CH


def _rownorm(x):
    ss = jnp.sum(x * x, axis=1, keepdims=True)
    nrm = jnp.sqrt(ss)
    return x / jnp.maximum(nrm, 1e-12)


def _loss_body(users_col, users_row, pos_col, pos_row,
               ue, pe, ne, ue0, pe0, ne0,
               u1, u2, i1, i2,
               mf_ref, cl_ref, reg_ref,
               mcol_s, mrow_s, n1s, n2s):
    uen = _rownorm(ue[...])
    pen = _rownorm(pe[...])
    nen = _rownorm(ne[...])
    pos_s = jnp.sum(uen * pen, axis=1, keepdims=True)
    neg_s = jnp.sum(uen * nen, axis=1, keepdims=True)
    x = pos_s - neg_s
    sig = 1.0 / (1.0 + jnp.exp(-x))
    maxi = jnp.log(sig + 1e-6)
    mf_ref[0, 0] = -jnp.sum(maxi) / _B

    reg = 0.5 * (jnp.sum(ue0[...] * ue0[...]) + jnp.sum(pe0[...] * pe0[...])
                 + jnp.sum(ne0[...] * ne0[...]))
    reg_ref[0, 0] = _DECAY * (reg / _B)

    def one_cl(ids_col, ids_row, v1, v2):
        def mask_col_body(c, carry):
            rows = ids_col[pl.ds(c * _CH, _CH), :]
            eq = rows == ids_row[...]
            ii = lax.broadcasted_iota(jnp.int32, (_CH, _B), 0) + c * _CH
            jj = lax.broadcasted_iota(jnp.int32, (_CH, _B), 1)
            dup = jnp.any(eq & (jj < ii), axis=1, keepdims=True)
            mcol_s[pl.ds(c * _CH, _CH), :] = jnp.where(dup, 0.0, 1.0)
            return carry

        lax.fori_loop(0, _NCH, mask_col_body, 0)

        def mask_row_body(c, carry):
            cols = ids_row[:, pl.ds(c * _CH, _CH)]
            eq = ids_col[...] == cols
            kk = lax.broadcasted_iota(jnp.int32, (_B, _CH), 0)
            jj = lax.broadcasted_iota(jnp.int32, (_B, _CH), 1) + c * _CH
            dup = jnp.any(eq & (kk < jj), axis=0, keepdims=True)
            mrow_s[:, pl.ds(c * _CH, _CH)] = jnp.where(dup, 0.0, 1.0)
            return carry

        lax.fori_loop(0, _NCH, mask_row_body, 0)

        count = jnp.sum(mcol_s[...])
        n1s[...] = _rownorm(v1[...])
        n2s[...] = _rownorm(v2[...])

        def chunk_body(c, acc):
            n1c = n1s[pl.ds(c * _CH, _CH), :]
            n2c = n2s[pl.ds(c * _CH, _CH), :]
            posv = jnp.exp(jnp.sum(n1c * n2c, axis=1, keepdims=True) / _TEMP)
            sm = lax.dot_general(n1c, n2s[...], (((1,), (1,)), ((), ())),
                                 preferred_element_type=jnp.float32)
            e = jnp.exp(sm / _TEMP) * mrow_s[...]
            ttl = jnp.sum(e, axis=1, keepdims=True)
            li = -jnp.log(posv / ttl + 1e-5)
            return acc + jnp.sum(li * mcol_s[pl.ds(c * _CH, _CH), :])

        tot = lax.fori_loop(0, _NCH, chunk_body, 0.0)
        return tot / count

    ucl = one_cl(users_col, users_row, u1, u2)
    icl = one_cl(pos_col, pos_row, i1, i2)
    cl_ref[0, 0] = _CL_RATE * (ucl + icl)


def _loss_call(users_col, users_row, pos_col, pos_row,
               ue, pe, ne, ue0, pe0, ne0, u1, u2, i1, i2):
    f32 = jnp.float32
    out = pl.pallas_call(
        _loss_body,
        out_shape=[jax.ShapeDtypeStruct((1, 1), f32)] * 3,
        out_specs=[pl.BlockSpec(memory_space=pltpu.SMEM)] * 3,
        scratch_shapes=[
            pltpu.VMEM((_B, 1), f32),
            pltpu.VMEM((1, _B), f32),
            pltpu.VMEM((_B, _EMB), f32),
            pltpu.VMEM((_B, _EMB), f32),
        ],
        interpret=_INTERPRET,
    )(users_col, users_row, pos_col, pos_row,
      ue, pe, ne, ue0, pe0, ne0, u1, u2, i1, i2)
    return out[0][0, 0], out[1][0, 0], out[2][0, 0]


def kernel(users, pos_items, neg_items, adj1_idx, adj1_val, adj2_idx, adj2_val,
           graph_idx, graph_val, embed_user, embed_item):
    f32 = jnp.float32
    i32 = jnp.int32

    all0 = jnp.concatenate(
        [embed_user, embed_item, jnp.zeros((_NPAD - _N, _EMB), f32)], axis=0)
    x0l = all0[:, :_HALF]
    x0h = all0[:, _HALF:]
    zeros = jnp.zeros((_ROWS_PER_TILE, _HALF), f32)

    npad = _NNZ_PAD - _NNZ

    def prep(idx, val):
        r = jnp.concatenate([idx[0].astype(i32), jnp.zeros((npad,), i32)])
        cc = jnp.concatenate([idx[1].astype(i32), jnp.zeros((npad,), i32)])
        v = jnp.concatenate([val, jnp.zeros((npad,), f32)])
        return (r.reshape(_IDX_ROWS, _SUB), cc.reshape(_IDX_ROWS, _SUB),
                v.reshape(_IDX_ROWS, _SUB))

    def chain(idx, val):
        rows2, cols2, vals2 = prep(idx, val)
        outs = []
        xl, xh = x0l, x0h
        for _ in range(_N_LAYERS):
            xl, xh = _spmm_call(xl, xh, rows2, cols2, vals2, zeros)
            outs.append((xl, xh))
        return outs

    g = chain(graph_idx, graph_val)
    a = chain(adj1_idx, adj1_val)
    b = chain(adj2_idx, adj2_val)

    users_i = users.astype(i32)
    pos_i = pos_items.astype(i32)
    neg_i = neg_items.astype(i32)
    uidx = users_i.reshape(_B // 128, 128)
    pidx = (pos_i + _N_USERS).reshape(_B // 128, 128)
    nidx = (neg_i + _N_USERS).reshape(_B // 128, 128)

    G = _gather_call(x0l, x0h,
                     g[0][0], g[0][1], g[1][0], g[1][1], g[2][0], g[2][1],
                     a[0][0], a[0][1], a[1][0], a[1][1], a[2][0], a[2][1],
                     b[0][0], b[0][1], b[1][0], b[1][1], b[2][0], b[2][1],
                     uidx, pidx, nidx)

    def full(q):
        return jnp.concatenate([G[0, q], G[1, q]], axis=-1)

    ue0, ue, u1, u2 = full(0), full(1), full(2), full(3)
    pe0, pe, i1, i2 = full(4), full(5), full(6), full(7)
    ne0, ne = full(8), full(9)

    users_col = users_i.reshape(_B, 1)
    users_row = users_i.reshape(1, _B)
    pos_col = pos_i.reshape(_B, 1)
    pos_row = pos_i.reshape(1, _B)

    mf, cl, reg = _loss_call(users_col, users_row, pos_col, pos_row,
                             ue, pe, ne, ue0, pe0, ne0, u1, u2, i1, i2)
    return mf, cl, reg


# speedup vs baseline: 4.0753x; 1.1751x over previous
import functools

import jax
import jax.numpy as jnp
from jax import lax
from jax.experimental import pallas as pl
from jax.experimental.pallas import tpu as pltpu
from jax.experimental.pallas import tpu_sc as plsc

_N_USERS = 20000
_N_ITEMS = 30000
_N = _N_USERS + _N_ITEMS
_EMB = 64
_HALF = 32
_NNZ = 1600000
_B = 4096
_N_LAYERS = 3
_CL_RATE = 0.2
_TEMP = 0.2
_DECAY = 1e-4

_INTERPRET = False

_NC = 2
_NS = 16

_NNZ_PAD = 1638400
_EDGES_PER_TILE = _NNZ_PAD // _NS
_KB = 256
_SUB = 128
_NSUB = _KB // _SUB
_NBLK = _EDGES_PER_TILE // _KB
_IDX_ROWS = _NNZ_PAD // _SUB
_NPAD = 50048
_ROWS_PER_TILE = _NPAD // _NS

_mesh = plsc.VectorSubcoreMesh(core_axis_name="c", subcore_axis_name="s",
                               num_cores=_NC, num_subcores=_NS)


def _spmm_body(x_lo, x_hi, rows2, cols2, vals2, zeros, y_lo, y_hi,
               ra, rb_, ca, cb, va, vb, xa, xb, acc, gsem, ssem):
    c = lax.axis_index("c")
    s = lax.axis_index("s")
    bufs = ((ra, ca, va, xa), (rb_, cb, vb, xb))

    def half(xsrc, ydst):
        pltpu.sync_copy(zeros, acc.at[pl.ds(s * _ROWS_PER_TILE, _ROWS_PER_TILE)])
        plsc.subcore_barrier()
        rbase = s * (_EDGES_PER_TILE // _SUB)

        def load_idx(b, bufp):
            r_, c_, v_, _ = bufp
            rr = rbase + b * _NSUB
            pltpu.sync_copy(rows2.at[pl.ds(rr, _NSUB)], r_)
            pltpu.sync_copy(cols2.at[pl.ds(rr, _NSUB)], c_)
            pltpu.sync_copy(vals2.at[pl.ds(rr, _NSUB)], v_)

        def fire_gather(bufp):
            _, c_, _, x_ = bufp
            for j in range(_NSUB):
                pltpu.async_copy(xsrc.at[c_.at[j]],
                                 x_.at[pl.ds(j * _SUB, _SUB)], gsem)

        def drain_gather(bufp):
            _, c_, _, x_ = bufp
            for j in range(_NSUB):
                pltpu.make_async_copy(xsrc.at[c_.at[j]],
                                      x_.at[pl.ds(j * _SUB, _SUB)], gsem).wait()

        def scale(bufp):
            _, _, v_, x_ = bufp
            for jj in range(_NSUB):
                def body(g, carry2, jj=jj):
                    vv = v_[jj, pl.ds(g * 16, 16)]
                    for l in range(16):
                        vs = vv[l]
                        r = jj * _SUB + g * 16 + l
                        x_[r, pl.ds(0, 16)] = x_[r, pl.ds(0, 16)] * vs
                        x_[r, pl.ds(16, 16)] = x_[r, pl.ds(16, 16)] * vs
                    return carry2
                lax.fori_loop(0, _SUB // 16, body, 0)

        def fire_scatter(bufp):
            r_, _, _, x_ = bufp
            for j in range(_NSUB):
                pltpu.async_copy(x_.at[pl.ds(j * _SUB, _SUB)],
                                 acc.at[r_.at[j]], ssem, add=True)

        def drain_scatter(bufp):
            r_, _, _, x_ = bufp
            for j in range(_NSUB):
                pltpu.make_async_copy(x_.at[pl.ds(j * _SUB, _SUB)],
                                      acc.at[r_.at[j]], ssem).wait()

        load_idx(0, bufs[0])
        fire_gather(bufs[0])

        def outer(g2, carry):
            for p in (0, 1):
                b = 2 * g2 + p
                cur, nxt = bufs[p], bufs[1 - p]
                if p == 0:
                    @pl.when(g2 > 0)
                    def _():
                        drain_scatter(nxt)
                else:
                    drain_scatter(nxt)

                @pl.when(b < _NBLK - 1)
                def _():
                    load_idx(b + 1, nxt)
                    fire_gather(nxt)

                drain_gather(cur)
                scale(cur)
                fire_scatter(cur)
            return carry

        lax.fori_loop(0, _NBLK // 2, outer, 0)
        drain_scatter(bufs[1])
        plsc.subcore_barrier()
        pltpu.sync_copy(acc.at[pl.ds(s * _ROWS_PER_TILE, _ROWS_PER_TILE)],
                        ydst.at[pl.ds(s * _ROWS_PER_TILE, _ROWS_PER_TILE)])

    @pl.when(c == 0)
    def _():
        half(x_lo, y_lo)

    @pl.when(c == 1)
    def _():
        half(x_hi, y_hi)


_spmm_call = pl.kernel(
    _spmm_body,
    out_type=[jax.ShapeDtypeStruct((_NPAD, _HALF), jnp.float32)] * 2,
    mesh=_mesh,
    scratch_types=[
        pltpu.VMEM((_NSUB, _SUB), jnp.int32),
        pltpu.VMEM((_NSUB, _SUB), jnp.int32),
        pltpu.VMEM((_NSUB, _SUB), jnp.int32),
        pltpu.VMEM((_NSUB, _SUB), jnp.int32),
        pltpu.VMEM((_NSUB, _SUB), jnp.float32),
        pltpu.VMEM((_NSUB, _SUB), jnp.float32),
        pltpu.VMEM((_KB, _HALF), jnp.float32),
        pltpu.VMEM((_KB, _HALF), jnp.float32),
        pltpu.VMEM_SHARED((_NPAD, _HALF), jnp.float32),
        pltpu.SemaphoreType.DMA,
        pltpu.SemaphoreType.DMA,
    ],
    compiler_params=pltpu.CompilerParams(use_tc_tiling_on_sc=False),
)


_GROWS = _B // _NS
_GSUB = _GROWS // 128


def _gather_body(x0l, x0h, g1l, g1h, g2l, g2h, g3l, g3h,
                 a1l, a1h, a2l, a2h, a3l, a3h,
                 b1l, b1h, b2l, b2h, b3l, b3h,
                 uidx, pidx, nidx, gout,
                 ibuf, b0, b1, b2, b3, mbuf, gsem):
    c = lax.axis_index("c")
    s = lax.axis_index("s")

    def run(hh, tabs):
        x0t, g1t, g2t, g3t, a1t, a2t, a3t, b1t, b2t, b3t = tabs

        def g4(t0, t1, t2, t3):
            descs = []
            for j in range(_GSUB):
                dst = pl.ds(j * 128, 128)
                idxr = ibuf.at[s * _GSUB + j]
                descs += [
                    pltpu.async_copy(t0.at[idxr], b0.at[dst], gsem),
                    pltpu.async_copy(t1.at[idxr], b1.at[dst], gsem),
                    pltpu.async_copy(t2.at[idxr], b2.at[dst], gsem),
                    pltpu.async_copy(t3.at[idxr], b3.at[dst], gsem),
                ]
            for d in descs:
                d.wait()

        def g3only(t1, t2, t3):
            descs = []
            for j in range(_GSUB):
                dst = pl.ds(j * 128, 128)
                idxr = ibuf.at[s * _GSUB + j]
                descs += [
                    pltpu.async_copy(t1.at[idxr], b1.at[dst], gsem),
                    pltpu.async_copy(t2.at[idxr], b2.at[dst], gsem),
                    pltpu.async_copy(t3.at[idxr], b3.at[dst], gsem),
                ]
            for d in descs:
                d.wait()

        def mean4():
            def mrow(r, carry):
                for h2 in range(2):
                    sl = pl.ds(h2 * 16, 16)
                    mbuf[r, sl] = (b0[r, sl] + b1[r, sl] + b2[r, sl]
                                   + b3[r, sl]) * 0.25
                return carry
            lax.fori_loop(0, _GROWS, mrow, 0, unroll=4)

        def wout(q, src):
            pltpu.sync_copy(src, gout.at[hh, q, pl.ds(s * _GROWS, _GROWS)])

        for idx_hbm, q0, has_cl in [(uidx, 0, True), (pidx, 4, True),
                                    (nidx, 8, False)]:
            pltpu.sync_copy(idx_hbm, ibuf)
            g4(x0t, g1t, g2t, g3t)
            wout(q0, b0)
            mean4()
            wout(q0 + 1, mbuf)
            if has_cl:
                g3only(a1t, a2t, a3t)
                mean4()
                wout(q0 + 2, mbuf)
                g3only(b1t, b2t, b3t)
                mean4()
                wout(q0 + 3, mbuf)

    @pl.when(c == 0)
    def _():
        run(0, (x0l, g1l, g2l, g3l, a1l, a2l, a3l, b1l, b2l, b3l))

    @pl.when(c == 1)
    def _():
        run(1, (x0h, g1h, g2h, g3h, a1h, a2h, a3h, b1h, b2h, b3h))


_gather_call = pl.kernel(
    _gather_body,
    out_type=jax.ShapeDtypeStruct((2, 10, _B, _HALF), jnp.float32),
    mesh=_mesh,
    scratch_types=[
        pltpu.VMEM((_B // 128, 128), jnp.int32),
        pltpu.VMEM((_GROWS, _HALF), jnp.float32),
        pltpu.VMEM((_GROWS, _HALF), jnp.float32),
        pltpu.VMEM((_GROWS, _HALF), jnp.float32),
        pltpu.VMEM((_GROWS, _HALF), jnp.float32),
        pltpu.VMEM((_GROWS, _HALF), jnp.float32),
        pltpu.SemaphoreType.DMA,
    ],
    compiler_params=pltpu.CompilerParams(use_tc_tiling_on_sc=False),
)


_CH = 512
_NCH = _B // _CH


def _rownorm(x):
    ss = jnp.sum(x * x, axis=1, keepdims=True)
    nrm = jnp.sqrt(ss)
    return x / jnp.maximum(nrm, 1e-12)


def _loss_body(users_col, users_row, pos_col, pos_row,
               ue, pe, ne, ue0, pe0, ne0,
               u1, u2, i1, i2,
               mf_ref, cl_ref, reg_ref,
               mcol_s, mrow_s, n1s, n2s):
    uen = _rownorm(ue[...])
    pen = _rownorm(pe[...])
    nen = _rownorm(ne[...])
    pos_s = jnp.sum(uen * pen, axis=1, keepdims=True)
    neg_s = jnp.sum(uen * nen, axis=1, keepdims=True)
    x = pos_s - neg_s
    sig = 1.0 / (1.0 + jnp.exp(-x))
    maxi = jnp.log(sig + 1e-6)
    mf_ref[0, 0] = -jnp.sum(maxi) / _B

    reg = 0.5 * (jnp.sum(ue0[...] * ue0[...]) + jnp.sum(pe0[...] * pe0[...])
                 + jnp.sum(ne0[...] * ne0[...]))
    reg_ref[0, 0] = _DECAY * (reg / _B)

    def one_cl(ids_col, ids_row, v1, v2):
        def mask_col_body(c, carry):
            rows = ids_col[pl.ds(c * _CH, _CH), :]
            eq = rows == ids_row[...]
            ii = lax.broadcasted_iota(jnp.int32, (_CH, _B), 0) + c * _CH
            jj = lax.broadcasted_iota(jnp.int32, (_CH, _B), 1)
            dup = jnp.any(eq & (jj < ii), axis=1, keepdims=True)
            mcol_s[pl.ds(c * _CH, _CH), :] = jnp.where(dup, 0.0, 1.0)
            return carry

        lax.fori_loop(0, _NCH, mask_col_body, 0)

        def mask_row_body(c, carry):
            cols = ids_row[:, pl.ds(c * _CH, _CH)]
            eq = ids_col[...] == cols
            kk = lax.broadcasted_iota(jnp.int32, (_B, _CH), 0)
            jj = lax.broadcasted_iota(jnp.int32, (_B, _CH), 1) + c * _CH
            dup = jnp.any(eq & (kk < jj), axis=0, keepdims=True)
            mrow_s[:, pl.ds(c * _CH, _CH)] = jnp.where(dup, 0.0, 1.0)
            return carry

        lax.fori_loop(0, _NCH, mask_row_body, 0)

        count = jnp.sum(mcol_s[...])
        n1s[...] = _rownorm(v1[...])
        n2s[...] = _rownorm(v2[...])

        def chunk_body(c, acc):
            n1c = n1s[pl.ds(c * _CH, _CH), :]
            n2c = n2s[pl.ds(c * _CH, _CH), :]
            posv = jnp.exp(jnp.sum(n1c * n2c, axis=1, keepdims=True) / _TEMP)
            sm = lax.dot_general(n1c, n2s[...], (((1,), (1,)), ((), ())),
                                 preferred_element_type=jnp.float32)
            e = jnp.exp(sm / _TEMP) * mrow_s[...]
            ttl = jnp.sum(e, axis=1, keepdims=True)
            li = -jnp.log(posv / ttl + 1e-5)
            return acc + jnp.sum(li * mcol_s[pl.ds(c * _CH, _CH), :])

        tot = lax.fori_loop(0, _NCH, chunk_body, 0.0)
        return tot / count

    ucl = one_cl(users_col, users_row, u1, u2)
    icl = one_cl(pos_col, pos_row, i1, i2)
    cl_ref[0, 0] = _CL_RATE * (ucl + icl)


def _loss_call(users_col, users_row, pos_col, pos_row,
               ue, pe, ne, ue0, pe0, ne0, u1, u2, i1, i2):
    f32 = jnp.float32
    out = pl.pallas_call(
        _loss_body,
        out_shape=[jax.ShapeDtypeStruct((1, 1), f32)] * 3,
        out_specs=[pl.BlockSpec(memory_space=pltpu.SMEM)] * 3,
        scratch_shapes=[
            pltpu.VMEM((_B, 1), f32),
            pltpu.VMEM((1, _B), f32),
            pltpu.VMEM((_B, _EMB), f32),
            pltpu.VMEM((_B, _EMB), f32),
        ],
        interpret=_INTERPRET,
    )(users_col, users_row, pos_col, pos_row,
      ue, pe, ne, ue0, pe0, ne0, u1, u2, i1, i2)
    return out[0][0, 0], out[1][0, 0], out[2][0, 0]


def kernel(users, pos_items, neg_items, adj1_idx, adj1_val, adj2_idx, adj2_val,
           graph_idx, graph_val, embed_user, embed_item):
    f32 = jnp.float32
    i32 = jnp.int32

    all0 = jnp.concatenate(
        [embed_user, embed_item, jnp.zeros((_NPAD - _N, _EMB), f32)], axis=0)
    x0l = all0[:, :_HALF]
    x0h = all0[:, _HALF:]
    zeros = jnp.zeros((_ROWS_PER_TILE, _HALF), f32)

    npad = _NNZ_PAD - _NNZ

    def prep(idx, val):
        r = jnp.concatenate([idx[0].astype(i32), jnp.zeros((npad,), i32)])
        cc = jnp.concatenate([idx[1].astype(i32), jnp.zeros((npad,), i32)])
        v = jnp.concatenate([val, jnp.zeros((npad,), f32)])
        return (r.reshape(_IDX_ROWS, _SUB), cc.reshape(_IDX_ROWS, _SUB),
                v.reshape(_IDX_ROWS, _SUB))

    def chain(idx, val):
        rows2, cols2, vals2 = prep(idx, val)
        outs = []
        xl, xh = x0l, x0h
        for _ in range(_N_LAYERS):
            xl, xh = _spmm_call(xl, xh, rows2, cols2, vals2, zeros)
            outs.append((xl, xh))
        return outs

    g = chain(graph_idx, graph_val)
    a = chain(adj1_idx, adj1_val)
    b = chain(adj2_idx, adj2_val)

    users_i = users.astype(i32)
    pos_i = pos_items.astype(i32)
    neg_i = neg_items.astype(i32)
    uidx = users_i.reshape(_B // 128, 128)
    pidx = (pos_i + _N_USERS).reshape(_B // 128, 128)
    nidx = (neg_i + _N_USERS).reshape(_B // 128, 128)

    G = _gather_call(x0l, x0h,
                     g[0][0], g[0][1], g[1][0], g[1][1], g[2][0], g[2][1],
                     a[0][0], a[0][1], a[1][0], a[1][1], a[2][0], a[2][1],
                     b[0][0], b[0][1], b[1][0], b[1][1], b[2][0], b[2][1],
                     uidx, pidx, nidx)

    def full(q):
        return jnp.concatenate([G[0, q], G[1, q]], axis=-1)

    ue0, ue, u1, u2 = full(0), full(1), full(2), full(3)
    pe0, pe, i1, i2 = full(4), full(5), full(6), full(7)
    ne0, ne = full(8), full(9)

    users_col = users_i.reshape(_B, 1)
    users_row = users_i.reshape(1, _B)
    pos_col = pos_i.reshape(_B, 1)
    pos_row = pos_i.reshape(1, _B)

    mf, cl, reg = _loss_call(users_col, users_row, pos_col, pos_row,
                             ue, pe, ne, ue0, pe0, ne0, u1, u2, i1, i2)
    return mf, cl, reg


# speedup vs baseline: 4.4082x; 1.0817x over previous
import functools

import jax
import jax.numpy as jnp
from jax import lax
from jax.experimental import pallas as pl
from jax.experimental.pallas import tpu as pltpu
from jax.experimental.pallas import tpu_sc as plsc

_N_USERS = 20000
_N_ITEMS = 30000
_N = _N_USERS + _N_ITEMS
_EMB = 64
_HALF = 32
_NNZ = 1600000
_B = 4096
_N_LAYERS = 3
_CL_RATE = 0.2
_TEMP = 0.2
_DECAY = 1e-4

_INTERPRET = False

_NC = 2
_NS = 16

_NNZ_PAD = 1638400
_EDGES_PER_TILE = _NNZ_PAD // _NS
_KB = 256
_SUB = 128
_NSUB = _KB // _SUB
_NBLK = _EDGES_PER_TILE // _KB
_IDX_ROWS = _NNZ_PAD // _SUB
_NPAD = 50048
_ROWS_PER_TILE = _NPAD // _NS

_mesh = plsc.VectorSubcoreMesh(core_axis_name="c", subcore_axis_name="s",
                               num_cores=_NC, num_subcores=_NS)


def _spmm_body(x_lo, x_hi, rows2, cols2, vals2, zeros, y_lo, y_hi,
               ra, rb_, ca, cb, va, vb, xa, xb, acc, gsem, ssem, isem):
    c = lax.axis_index("c")
    s = lax.axis_index("s")
    bufs = ((ra, ca, va, xa), (rb_, cb, vb, xb))

    def half(xsrc, ydst):
        pltpu.sync_copy(zeros, acc.at[pl.ds(s * _ROWS_PER_TILE, _ROWS_PER_TILE)])
        plsc.subcore_barrier()
        rbase = s * (_EDGES_PER_TILE // _SUB)

        def fire_idx(b, bufp):
            r_, c_, v_, _ = bufp
            rr = rbase + b * _NSUB
            pltpu.async_copy(rows2.at[pl.ds(rr, _NSUB)], r_, isem)
            pltpu.async_copy(cols2.at[pl.ds(rr, _NSUB)], c_, isem)
            pltpu.async_copy(vals2.at[pl.ds(rr, _NSUB)], v_, isem)

        def drain_idx():
            pltpu.make_async_copy(rows2.at[pl.ds(0, _NSUB)], ra, isem).wait()
            pltpu.make_async_copy(cols2.at[pl.ds(0, _NSUB)], ca, isem).wait()
            pltpu.make_async_copy(vals2.at[pl.ds(0, _NSUB)], va, isem).wait()

        def fire_gather(bufp):
            _, c_, _, x_ = bufp
            for j in range(_NSUB):
                pltpu.async_copy(xsrc.at[c_.at[j]],
                                 x_.at[pl.ds(j * _SUB, _SUB)], gsem)

        def drain_gather():
            for j in range(_NSUB):
                pltpu.make_async_copy(xsrc.at[ca.at[j]],
                                      xa.at[pl.ds(j * _SUB, _SUB)], gsem).wait()

        def scale(bufp):
            _, _, v_, x_ = bufp
            for jj in range(_NSUB):
                def body(g, carry2, jj=jj):
                    vv = v_[jj, pl.ds(g * 16, 16)]
                    for l in range(16):
                        vs = vv[l]
                        r = jj * _SUB + g * 16 + l
                        x_[r, pl.ds(0, 16)] = x_[r, pl.ds(0, 16)] * vs
                        x_[r, pl.ds(16, 16)] = x_[r, pl.ds(16, 16)] * vs
                    return carry2
                lax.fori_loop(0, _SUB // 16, body, 0)

        def fire_scatter(bufp):
            r_, _, _, x_ = bufp
            for j in range(_NSUB):
                pltpu.async_copy(x_.at[pl.ds(j * _SUB, _SUB)],
                                 acc.at[r_.at[j]], ssem, add=True)

        def drain_scatter():
            for j in range(_NSUB):
                pltpu.make_async_copy(xa.at[pl.ds(j * _SUB, _SUB)],
                                      acc.at[ra.at[j]], ssem).wait()

        fire_idx(0, bufs[0])
        fire_idx(1, bufs[1])
        drain_idx()
        fire_gather(bufs[0])

        def outer(g2, carry):
            for p in (0, 1):
                b = 2 * g2 + p
                cur, nxt = bufs[p], bufs[1 - p]

                @pl.when(b >= 1)
                def _():
                    drain_scatter()

                drain_gather()
                scale(cur)
                fire_scatter(cur)

                @pl.when(b < _NBLK - 1)
                def _():
                    drain_idx()
                    fire_gather(nxt)

                @pl.when(b < _NBLK - 2)
                def _():
                    fire_idx(b + 2, cur)
            return carry

        lax.fori_loop(0, _NBLK // 2, outer, 0)
        drain_scatter()
        plsc.subcore_barrier()
        pltpu.sync_copy(acc.at[pl.ds(s * _ROWS_PER_TILE, _ROWS_PER_TILE)],
                        ydst.at[pl.ds(s * _ROWS_PER_TILE, _ROWS_PER_TILE)])

    @pl.when(c == 0)
    def _():
        half(x_lo, y_lo)

    @pl.when(c == 1)
    def _():
        half(x_hi, y_hi)


_spmm_call = pl.kernel(
    _spmm_body,
    out_type=[jax.ShapeDtypeStruct((_NPAD, _HALF), jnp.float32)] * 2,
    mesh=_mesh,
    scratch_types=[
        pltpu.VMEM((_NSUB, _SUB), jnp.int32),
        pltpu.VMEM((_NSUB, _SUB), jnp.int32),
        pltpu.VMEM((_NSUB, _SUB), jnp.int32),
        pltpu.VMEM((_NSUB, _SUB), jnp.int32),
        pltpu.VMEM((_NSUB, _SUB), jnp.float32),
        pltpu.VMEM((_NSUB, _SUB), jnp.float32),
        pltpu.VMEM((_KB, _HALF), jnp.float32),
        pltpu.VMEM((_KB, _HALF), jnp.float32),
        pltpu.VMEM_SHARED((_NPAD, _HALF), jnp.float32),
        pltpu.SemaphoreType.DMA,
        pltpu.SemaphoreType.DMA,
        pltpu.SemaphoreType.DMA,
    ],
    compiler_params=pltpu.CompilerParams(use_tc_tiling_on_sc=False),
)


_GROWS = _B // _NS
_GSUB = _GROWS // 128


def _gather_body(x0l, x0h, g1l, g1h, g2l, g2h, g3l, g3h,
                 a1l, a1h, a2l, a2h, a3l, a3h,
                 b1l, b1h, b2l, b2h, b3l, b3h,
                 uidx, pidx, nidx, gout,
                 ibuf, b0, b1, b2, b3, mbuf, gsem):
    c = lax.axis_index("c")
    s = lax.axis_index("s")

    def run(hh, tabs):
        x0t, g1t, g2t, g3t, a1t, a2t, a3t, b1t, b2t, b3t = tabs

        def g4(t0, t1, t2, t3):
            descs = []
            for j in range(_GSUB):
                dst = pl.ds(j * 128, 128)
                idxr = ibuf.at[s * _GSUB + j]
                descs += [
                    pltpu.async_copy(t0.at[idxr], b0.at[dst], gsem),
                    pltpu.async_copy(t1.at[idxr], b1.at[dst], gsem),
                    pltpu.async_copy(t2.at[idxr], b2.at[dst], gsem),
                    pltpu.async_copy(t3.at[idxr], b3.at[dst], gsem),
                ]
            for d in descs:
                d.wait()

        def g3only(t1, t2, t3):
            descs = []
            for j in range(_GSUB):
                dst = pl.ds(j * 128, 128)
                idxr = ibuf.at[s * _GSUB + j]
                descs += [
                    pltpu.async_copy(t1.at[idxr], b1.at[dst], gsem),
                    pltpu.async_copy(t2.at[idxr], b2.at[dst], gsem),
                    pltpu.async_copy(t3.at[idxr], b3.at[dst], gsem),
                ]
            for d in descs:
                d.wait()

        def mean4():
            def mrow(r, carry):
                for h2 in range(2):
                    sl = pl.ds(h2 * 16, 16)
                    mbuf[r, sl] = (b0[r, sl] + b1[r, sl] + b2[r, sl]
                                   + b3[r, sl]) * 0.25
                return carry
            lax.fori_loop(0, _GROWS, mrow, 0, unroll=4)

        def wout(q, src):
            pltpu.sync_copy(src, gout.at[hh, q, pl.ds(s * _GROWS, _GROWS)])

        for idx_hbm, q0, has_cl in [(uidx, 0, True), (pidx, 4, True),
                                    (nidx, 8, False)]:
            pltpu.sync_copy(idx_hbm, ibuf)
            g4(x0t, g1t, g2t, g3t)
            wout(q0, b0)
            mean4()
            wout(q0 + 1, mbuf)
            if has_cl:
                g3only(a1t, a2t, a3t)
                mean4()
                wout(q0 + 2, mbuf)
                g3only(b1t, b2t, b3t)
                mean4()
                wout(q0 + 3, mbuf)

    @pl.when(c == 0)
    def _():
        run(0, (x0l, g1l, g2l, g3l, a1l, a2l, a3l, b1l, b2l, b3l))

    @pl.when(c == 1)
    def _():
        run(1, (x0h, g1h, g2h, g3h, a1h, a2h, a3h, b1h, b2h, b3h))


_gather_call = pl.kernel(
    _gather_body,
    out_type=jax.ShapeDtypeStruct((2, 10, _B, _HALF), jnp.float32),
    mesh=_mesh,
    scratch_types=[
        pltpu.VMEM((_B // 128, 128), jnp.int32),
        pltpu.VMEM((_GROWS, _HALF), jnp.float32),
        pltpu.VMEM((_GROWS, _HALF), jnp.float32),
        pltpu.VMEM((_GROWS, _HALF), jnp.float32),
        pltpu.VMEM((_GROWS, _HALF), jnp.float32),
        pltpu.VMEM((_GROWS, _HALF), jnp.float32),
        pltpu.SemaphoreType.DMA,
    ],
    compiler_params=pltpu.CompilerParams(use_tc_tiling_on_sc=False),
)


_CH = 512
_NCH = _B // _CH


def _rownorm(x):
    ss = jnp.sum(x * x, axis=1, keepdims=True)
    nrm = jnp.sqrt(ss)
    return x / jnp.maximum(nrm, 1e-12)


def _loss_body(users_col, users_row, pos_col, pos_row,
               ue, pe, ne, ue0, pe0, ne0,
               u1, u2, i1, i2,
               mf_ref, cl_ref, reg_ref,
               mcol_s, mrow_s, n1s, n2s):
    uen = _rownorm(ue[...])
    pen = _rownorm(pe[...])
    nen = _rownorm(ne[...])
    pos_s = jnp.sum(uen * pen, axis=1, keepdims=True)
    neg_s = jnp.sum(uen * nen, axis=1, keepdims=True)
    x = pos_s - neg_s
    sig = 1.0 / (1.0 + jnp.exp(-x))
    maxi = jnp.log(sig + 1e-6)
    mf_ref[0, 0] = -jnp.sum(maxi) / _B

    reg = 0.5 * (jnp.sum(ue0[...] * ue0[...]) + jnp.sum(pe0[...] * pe0[...])
                 + jnp.sum(ne0[...] * ne0[...]))
    reg_ref[0, 0] = _DECAY * (reg / _B)

    def one_cl(ids_col, ids_row, v1, v2):
        def mask_col_body(c, carry):
            rows = ids_col[pl.ds(c * _CH, _CH), :]
            eq = rows == ids_row[...]
            ii = lax.broadcasted_iota(jnp.int32, (_CH, _B), 0) + c * _CH
            jj = lax.broadcasted_iota(jnp.int32, (_CH, _B), 1)
            dup = jnp.any(eq & (jj < ii), axis=1, keepdims=True)
            mcol_s[pl.ds(c * _CH, _CH), :] = jnp.where(dup, 0.0, 1.0)
            return carry

        lax.fori_loop(0, _NCH, mask_col_body, 0)

        def mask_row_body(c, carry):
            cols = ids_row[:, pl.ds(c * _CH, _CH)]
            eq = ids_col[...] == cols
            kk = lax.broadcasted_iota(jnp.int32, (_B, _CH), 0)
            jj = lax.broadcasted_iota(jnp.int32, (_B, _CH), 1) + c * _CH
            dup = jnp.any(eq & (kk < jj), axis=0, keepdims=True)
            mrow_s[:, pl.ds(c * _CH, _CH)] = jnp.where(dup, 0.0, 1.0)
            return carry

        lax.fori_loop(0, _NCH, mask_row_body, 0)

        count = jnp.sum(mcol_s[...])
        n1s[...] = _rownorm(v1[...])
        n2s[...] = _rownorm(v2[...])

        def chunk_body(c, acc):
            n1c = n1s[pl.ds(c * _CH, _CH), :]
            n2c = n2s[pl.ds(c * _CH, _CH), :]
            posv = jnp.exp(jnp.sum(n1c * n2c, axis=1, keepdims=True) / _TEMP)
            sm = lax.dot_general(n1c, n2s[...], (((1,), (1,)), ((), ())),
                                 preferred_element_type=jnp.float32)
            e = jnp.exp(sm / _TEMP) * mrow_s[...]
            ttl = jnp.sum(e, axis=1, keepdims=True)
            li = -jnp.log(posv / ttl + 1e-5)
            return acc + jnp.sum(li * mcol_s[pl.ds(c * _CH, _CH), :])

        tot = lax.fori_loop(0, _NCH, chunk_body, 0.0)
        return tot / count

    ucl = one_cl(users_col, users_row, u1, u2)
    icl = one_cl(pos_col, pos_row, i1, i2)
    cl_ref[0, 0] = _CL_RATE * (ucl + icl)


def _loss_call(users_col, users_row, pos_col, pos_row,
               ue, pe, ne, ue0, pe0, ne0, u1, u2, i1, i2):
    f32 = jnp.float32
    out = pl.pallas_call(
        _loss_body,
        out_shape=[jax.ShapeDtypeStruct((1, 1), f32)] * 3,
        out_specs=[pl.BlockSpec(memory_space=pltpu.SMEM)] * 3,
        scratch_shapes=[
            pltpu.VMEM((_B, 1), f32),
            pltpu.VMEM((1, _B), f32),
            pltpu.VMEM((_B, _EMB), f32),
            pltpu.VMEM((_B, _EMB), f32),
        ],
        interpret=_INTERPRET,
    )(users_col, users_row, pos_col, pos_row,
      ue, pe, ne, ue0, pe0, ne0, u1, u2, i1, i2)
    return out[0][0, 0], out[1][0, 0], out[2][0, 0]


def kernel(users, pos_items, neg_items, adj1_idx, adj1_val, adj2_idx, adj2_val,
           graph_idx, graph_val, embed_user, embed_item):
    f32 = jnp.float32
    i32 = jnp.int32

    all0 = jnp.concatenate(
        [embed_user, embed_item, jnp.zeros((_NPAD - _N, _EMB), f32)], axis=0)
    x0l = all0[:, :_HALF]
    x0h = all0[:, _HALF:]
    zeros = jnp.zeros((_ROWS_PER_TILE, _HALF), f32)

    npad = _NNZ_PAD - _NNZ

    def prep(idx, val):
        r = jnp.concatenate([idx[0].astype(i32), jnp.zeros((npad,), i32)])
        cc = jnp.concatenate([idx[1].astype(i32), jnp.zeros((npad,), i32)])
        v = jnp.concatenate([val, jnp.zeros((npad,), f32)])
        return (r.reshape(_IDX_ROWS, _SUB), cc.reshape(_IDX_ROWS, _SUB),
                v.reshape(_IDX_ROWS, _SUB))

    def chain(idx, val):
        rows2, cols2, vals2 = prep(idx, val)
        outs = []
        xl, xh = x0l, x0h
        for _ in range(_N_LAYERS):
            xl, xh = _spmm_call(xl, xh, rows2, cols2, vals2, zeros)
            outs.append((xl, xh))
        return outs

    g = chain(graph_idx, graph_val)
    a = chain(adj1_idx, adj1_val)
    b = chain(adj2_idx, adj2_val)

    users_i = users.astype(i32)
    pos_i = pos_items.astype(i32)
    neg_i = neg_items.astype(i32)
    uidx = users_i.reshape(_B // 128, 128)
    pidx = (pos_i + _N_USERS).reshape(_B // 128, 128)
    nidx = (neg_i + _N_USERS).reshape(_B // 128, 128)

    G = _gather_call(x0l, x0h,
                     g[0][0], g[0][1], g[1][0], g[1][1], g[2][0], g[2][1],
                     a[0][0], a[0][1], a[1][0], a[1][1], a[2][0], a[2][1],
                     b[0][0], b[0][1], b[1][0], b[1][1], b[2][0], b[2][1],
                     uidx, pidx, nidx)

    def full(q):
        return jnp.concatenate([G[0, q], G[1, q]], axis=-1)

    ue0, ue, u1, u2 = full(0), full(1), full(2), full(3)
    pe0, pe, i1, i2 = full(4), full(5), full(6), full(7)
    ne0, ne = full(8), full(9)

    users_col = users_i.reshape(_B, 1)
    users_row = users_i.reshape(1, _B)
    pos_col = pos_i.reshape(_B, 1)
    pos_row = pos_i.reshape(1, _B)

    mf, cl, reg = _loss_call(users_col, users_row, pos_col, pos_row,
                             ue, pe, ne, ue0, pe0, ne0, u1, u2, i1, i2)
    return mf, cl, reg


# speedup vs baseline: 4.5874x; 1.0407x over previous
import functools

import jax
import jax.numpy as jnp
from jax import lax
from jax.experimental import pallas as pl
from jax.experimental.pallas import tpu as pltpu
from jax.experimental.pallas import tpu_sc as plsc

_N_USERS = 20000
_N_ITEMS = 30000
_N = _N_USERS + _N_ITEMS
_EMB = 64
_HALF = 32
_NNZ = 1600000
_B = 4096
_N_LAYERS = 3
_CL_RATE = 0.2
_TEMP = 0.2
_DECAY = 1e-4

_INTERPRET = False

_NC = 2
_NS = 16

_NNZ_PAD = 1638400
_EDGES_PER_TILE = _NNZ_PAD // _NS
_KB = 320
_NBLK = _EDGES_PER_TILE // _KB
_NPAD = 50048
_ROWS_PER_TILE = _NPAD // _NS

_mesh = plsc.VectorSubcoreMesh(core_axis_name="c", subcore_axis_name="s",
                               num_cores=_NC, num_subcores=_NS)


def _spmm_body(x_lo, x_hi, rows1, cols1, vals1, zeros, y_lo, y_hi,
               ra, rb_, ca, cb, va, vb, xa, xb, acc, gsem, ssem, isem):
    c = lax.axis_index("c")
    s = lax.axis_index("s")
    bufs = ((ra, ca, va, xa), (rb_, cb, vb, xb))

    def half(xsrc, ydst):
        pltpu.sync_copy(zeros, acc.at[pl.ds(s * _ROWS_PER_TILE, _ROWS_PER_TILE)])
        plsc.subcore_barrier()
        ebase = s * _EDGES_PER_TILE

        def fire_idx(b, bufp):
            r_, c_, v_, _ = bufp
            ee = ebase + b * _KB
            pltpu.async_copy(rows1.at[pl.ds(ee, _KB)], r_, isem)
            pltpu.async_copy(cols1.at[pl.ds(ee, _KB)], c_, isem)
            pltpu.async_copy(vals1.at[pl.ds(ee, _KB)], v_, isem)

        def drain_idx():
            pltpu.make_async_copy(rows1.at[pl.ds(0, _KB)], ra, isem).wait()
            pltpu.make_async_copy(cols1.at[pl.ds(0, _KB)], ca, isem).wait()
            pltpu.make_async_copy(vals1.at[pl.ds(0, _KB)], va, isem).wait()

        def fire_gather(bufp):
            _, c_, _, x_ = bufp
            pltpu.async_copy(xsrc.at[c_], x_, gsem)

        def drain_gather():
            pltpu.make_async_copy(xsrc.at[ca], xa, gsem).wait()

        def scale(bufp):
            _, _, v_, x_ = bufp

            def body(g, carry2):
                vv = v_[pl.ds(g * 16, 16)]
                for l in range(16):
                    vs = vv[l]
                    r = g * 16 + l
                    x_[r, pl.ds(0, 16)] = x_[r, pl.ds(0, 16)] * vs
                    x_[r, pl.ds(16, 16)] = x_[r, pl.ds(16, 16)] * vs
                return carry2

            lax.fori_loop(0, _KB // 16, body, 0)

        def fire_scatter(bufp):
            r_, _, _, x_ = bufp
            pltpu.async_copy(x_, acc.at[r_], ssem, add=True)

        def drain_scatter():
            pltpu.make_async_copy(xa, acc.at[ra], ssem).wait()

        fire_idx(0, bufs[0])
        fire_idx(1, bufs[1])
        drain_idx()
        fire_gather(bufs[0])

        def outer(g2, carry):
            for p in (0, 1):
                b = 2 * g2 + p
                cur, nxt = bufs[p], bufs[1 - p]

                @pl.when(b >= 1)
                def _():
                    drain_scatter()

                drain_gather()
                scale(cur)
                fire_scatter(cur)

                @pl.when(b < _NBLK - 1)
                def _():
                    drain_idx()
                    fire_gather(nxt)

                @pl.when(b < _NBLK - 2)
                def _():
                    fire_idx(b + 2, cur)
            return carry

        lax.fori_loop(0, _NBLK // 2, outer, 0)
        drain_scatter()
        plsc.subcore_barrier()
        pltpu.sync_copy(acc.at[pl.ds(s * _ROWS_PER_TILE, _ROWS_PER_TILE)],
                        ydst.at[pl.ds(s * _ROWS_PER_TILE, _ROWS_PER_TILE)])

    @pl.when(c == 0)
    def _():
        half(x_lo, y_lo)

    @pl.when(c == 1)
    def _():
        half(x_hi, y_hi)


_spmm_call = pl.kernel(
    _spmm_body,
    out_type=[jax.ShapeDtypeStruct((_NPAD, _HALF), jnp.float32)] * 2,
    mesh=_mesh,
    scratch_types=[
        pltpu.VMEM((_KB,), jnp.int32),
        pltpu.VMEM((_KB,), jnp.int32),
        pltpu.VMEM((_KB,), jnp.int32),
        pltpu.VMEM((_KB,), jnp.int32),
        pltpu.VMEM((_KB,), jnp.float32),
        pltpu.VMEM((_KB,), jnp.float32),
        pltpu.VMEM((_KB, _HALF), jnp.float32),
        pltpu.VMEM((_KB, _HALF), jnp.float32),
        pltpu.VMEM_SHARED((_NPAD, _HALF), jnp.float32),
        pltpu.SemaphoreType.DMA,
        pltpu.SemaphoreType.DMA,
        pltpu.SemaphoreType.DMA,
    ],
    compiler_params=pltpu.CompilerParams(use_tc_tiling_on_sc=False),
)


_GROWS = _B // _NS
_GSUB = _GROWS // 128


def _gather_body(x0l, x0h, g1l, g1h, g2l, g2h, g3l, g3h,
                 a1l, a1h, a2l, a2h, a3l, a3h,
                 b1l, b1h, b2l, b2h, b3l, b3h,
                 uidx, pidx, nidx, gout,
                 ibuf, b0, b1, b2, b3, mbuf, gsem):
    c = lax.axis_index("c")
    s = lax.axis_index("s")

    def run(hh, tabs):
        x0t, g1t, g2t, g3t, a1t, a2t, a3t, b1t, b2t, b3t = tabs

        def g4(t0, t1, t2, t3):
            descs = []
            for j in range(_GSUB):
                dst = pl.ds(j * 128, 128)
                idxr = ibuf.at[s * _GSUB + j]
                descs += [
                    pltpu.async_copy(t0.at[idxr], b0.at[dst], gsem),
                    pltpu.async_copy(t1.at[idxr], b1.at[dst], gsem),
                    pltpu.async_copy(t2.at[idxr], b2.at[dst], gsem),
                    pltpu.async_copy(t3.at[idxr], b3.at[dst], gsem),
                ]
            for d in descs:
                d.wait()

        def g3only(t1, t2, t3):
            descs = []
            for j in range(_GSUB):
                dst = pl.ds(j * 128, 128)
                idxr = ibuf.at[s * _GSUB + j]
                descs += [
                    pltpu.async_copy(t1.at[idxr], b1.at[dst], gsem),
                    pltpu.async_copy(t2.at[idxr], b2.at[dst], gsem),
                    pltpu.async_copy(t3.at[idxr], b3.at[dst], gsem),
                ]
            for d in descs:
                d.wait()

        def mean4():
            def mrow(r, carry):
                for h2 in range(2):
                    sl = pl.ds(h2 * 16, 16)
                    mbuf[r, sl] = (b0[r, sl] + b1[r, sl] + b2[r, sl]
                                   + b3[r, sl]) * 0.25
                return carry
            lax.fori_loop(0, _GROWS, mrow, 0, unroll=4)

        def wout(q, src):
            pltpu.sync_copy(src, gout.at[hh, q, pl.ds(s * _GROWS, _GROWS)])

        for idx_hbm, q0, has_cl in [(uidx, 0, True), (pidx, 4, True),
                                    (nidx, 8, False)]:
            pltpu.sync_copy(idx_hbm, ibuf)
            g4(x0t, g1t, g2t, g3t)
            wout(q0, b0)
            mean4()
            wout(q0 + 1, mbuf)
            if has_cl:
                g3only(a1t, a2t, a3t)
                mean4()
                wout(q0 + 2, mbuf)
                g3only(b1t, b2t, b3t)
                mean4()
                wout(q0 + 3, mbuf)

    @pl.when(c == 0)
    def _():
        run(0, (x0l, g1l, g2l, g3l, a1l, a2l, a3l, b1l, b2l, b3l))

    @pl.when(c == 1)
    def _():
        run(1, (x0h, g1h, g2h, g3h, a1h, a2h, a3h, b1h, b2h, b3h))


_gather_call = pl.kernel(
    _gather_body,
    out_type=jax.ShapeDtypeStruct((2, 10, _B, _HALF), jnp.float32),
    mesh=_mesh,
    scratch_types=[
        pltpu.VMEM((_B // 128, 128), jnp.int32),
        pltpu.VMEM((_GROWS, _HALF), jnp.float32),
        pltpu.VMEM((_GROWS, _HALF), jnp.float32),
        pltpu.VMEM((_GROWS, _HALF), jnp.float32),
        pltpu.VMEM((_GROWS, _HALF), jnp.float32),
        pltpu.VMEM((_GROWS, _HALF), jnp.float32),
        pltpu.SemaphoreType.DMA,
    ],
    compiler_params=pltpu.CompilerParams(use_tc_tiling_on_sc=False),
)


_CH = 512
_NCH = _B // _CH


def _rownorm(x):
    ss = jnp.sum(x * x, axis=1, keepdims=True)
    nrm = jnp.sqrt(ss)
    return x / jnp.maximum(nrm, 1e-12)


def _loss_body(users_col, users_row, pos_col, pos_row,
               ue, pe, ne, ue0, pe0, ne0,
               u1, u2, i1, i2,
               mf_ref, cl_ref, reg_ref,
               mcol_s, mrow_s, n1s, n2s):
    uen = _rownorm(ue[...])
    pen = _rownorm(pe[...])
    nen = _rownorm(ne[...])
    pos_s = jnp.sum(uen * pen, axis=1, keepdims=True)
    neg_s = jnp.sum(uen * nen, axis=1, keepdims=True)
    x = pos_s - neg_s
    sig = 1.0 / (1.0 + jnp.exp(-x))
    maxi = jnp.log(sig + 1e-6)
    mf_ref[0, 0] = -jnp.sum(maxi) / _B

    reg = 0.5 * (jnp.sum(ue0[...] * ue0[...]) + jnp.sum(pe0[...] * pe0[...])
                 + jnp.sum(ne0[...] * ne0[...]))
    reg_ref[0, 0] = _DECAY * (reg / _B)

    def one_cl(ids_col, ids_row, v1, v2):
        def mask_col_body(c, carry):
            rows = ids_col[pl.ds(c * _CH, _CH), :]
            eq = rows == ids_row[...]
            ii = lax.broadcasted_iota(jnp.int32, (_CH, _B), 0) + c * _CH
            jj = lax.broadcasted_iota(jnp.int32, (_CH, _B), 1)
            dup = jnp.any(eq & (jj < ii), axis=1, keepdims=True)
            mcol_s[pl.ds(c * _CH, _CH), :] = jnp.where(dup, 0.0, 1.0)
            return carry

        lax.fori_loop(0, _NCH, mask_col_body, 0)

        def mask_row_body(c, carry):
            cols = ids_row[:, pl.ds(c * _CH, _CH)]
            eq = ids_col[...] == cols
            kk = lax.broadcasted_iota(jnp.int32, (_B, _CH), 0)
            jj = lax.broadcasted_iota(jnp.int32, (_B, _CH), 1) + c * _CH
            dup = jnp.any(eq & (kk < jj), axis=0, keepdims=True)
            mrow_s[:, pl.ds(c * _CH, _CH)] = jnp.where(dup, 0.0, 1.0)
            return carry

        lax.fori_loop(0, _NCH, mask_row_body, 0)

        count = jnp.sum(mcol_s[...])
        n1s[...] = _rownorm(v1[...])
        n2s[...] = _rownorm(v2[...])

        def chunk_body(c, acc):
            n1c = n1s[pl.ds(c * _CH, _CH), :]
            n2c = n2s[pl.ds(c * _CH, _CH), :]
            posv = jnp.exp(jnp.sum(n1c * n2c, axis=1, keepdims=True) / _TEMP)
            sm = lax.dot_general(n1c, n2s[...], (((1,), (1,)), ((), ())),
                                 preferred_element_type=jnp.float32)
            e = jnp.exp(sm / _TEMP) * mrow_s[...]
            ttl = jnp.sum(e, axis=1, keepdims=True)
            li = -jnp.log(posv / ttl + 1e-5)
            return acc + jnp.sum(li * mcol_s[pl.ds(c * _CH, _CH), :])

        tot = lax.fori_loop(0, _NCH, chunk_body, 0.0)
        return tot / count

    ucl = one_cl(users_col, users_row, u1, u2)
    icl = one_cl(pos_col, pos_row, i1, i2)
    cl_ref[0, 0] = _CL_RATE * (ucl + icl)


def _loss_call(users_col, users_row, pos_col, pos_row,
               ue, pe, ne, ue0, pe0, ne0, u1, u2, i1, i2):
    f32 = jnp.float32
    out = pl.pallas_call(
        _loss_body,
        out_shape=[jax.ShapeDtypeStruct((1, 1), f32)] * 3,
        out_specs=[pl.BlockSpec(memory_space=pltpu.SMEM)] * 3,
        scratch_shapes=[
            pltpu.VMEM((_B, 1), f32),
            pltpu.VMEM((1, _B), f32),
            pltpu.VMEM((_B, _EMB), f32),
            pltpu.VMEM((_B, _EMB), f32),
        ],
        interpret=_INTERPRET,
    )(users_col, users_row, pos_col, pos_row,
      ue, pe, ne, ue0, pe0, ne0, u1, u2, i1, i2)
    return out[0][0, 0], out[1][0, 0], out[2][0, 0]


def kernel(users, pos_items, neg_items, adj1_idx, adj1_val, adj2_idx, adj2_val,
           graph_idx, graph_val, embed_user, embed_item):
    f32 = jnp.float32
    i32 = jnp.int32

    all0 = jnp.concatenate(
        [embed_user, embed_item, jnp.zeros((_NPAD - _N, _EMB), f32)], axis=0)
    x0l = all0[:, :_HALF]
    x0h = all0[:, _HALF:]
    zeros = jnp.zeros((_ROWS_PER_TILE, _HALF), f32)

    npad = _NNZ_PAD - _NNZ

    def prep(idx, val):
        r = jnp.concatenate([idx[0].astype(i32), jnp.zeros((npad,), i32)])
        cc = jnp.concatenate([idx[1].astype(i32), jnp.zeros((npad,), i32)])
        v = jnp.concatenate([val, jnp.zeros((npad,), f32)])
        return r, cc, v

    def chain(idx, val):
        rows2, cols2, vals2 = prep(idx, val)
        outs = []
        xl, xh = x0l, x0h
        for _ in range(_N_LAYERS):
            xl, xh = _spmm_call(xl, xh, rows2, cols2, vals2, zeros)
            outs.append((xl, xh))
        return outs

    g = chain(graph_idx, graph_val)
    a = chain(adj1_idx, adj1_val)
    b = chain(adj2_idx, adj2_val)

    users_i = users.astype(i32)
    pos_i = pos_items.astype(i32)
    neg_i = neg_items.astype(i32)
    uidx = users_i.reshape(_B // 128, 128)
    pidx = (pos_i + _N_USERS).reshape(_B // 128, 128)
    nidx = (neg_i + _N_USERS).reshape(_B // 128, 128)

    G = _gather_call(x0l, x0h,
                     g[0][0], g[0][1], g[1][0], g[1][1], g[2][0], g[2][1],
                     a[0][0], a[0][1], a[1][0], a[1][1], a[2][0], a[2][1],
                     b[0][0], b[0][1], b[1][0], b[1][1], b[2][0], b[2][1],
                     uidx, pidx, nidx)

    def full(q):
        return jnp.concatenate([G[0, q], G[1, q]], axis=-1)

    ue0, ue, u1, u2 = full(0), full(1), full(2), full(3)
    pe0, pe, i1, i2 = full(4), full(5), full(6), full(7)
    ne0, ne = full(8), full(9)

    users_col = users_i.reshape(_B, 1)
    users_row = users_i.reshape(1, _B)
    pos_col = pos_i.reshape(_B, 1)
    pos_row = pos_i.reshape(1, _B)

    mf, cl, reg = _loss_call(users_col, users_row, pos_col, pos_row,
                             ue, pe, ne, ue0, pe0, ne0, u1, u2, i1, i2)
    return mf, cl, reg


# speedup vs baseline: 5.5457x; 1.2089x over previous
import functools

import jax
import jax.numpy as jnp
from jax import lax
from jax.experimental import pallas as pl
from jax.experimental.pallas import tpu as pltpu
from jax.experimental.pallas import tpu_sc as plsc

_N_USERS = 20000
_N_ITEMS = 30000
_N = _N_USERS + _N_ITEMS
_EMB = 64
_HALF = 32
_NNZ = 1600000
_B = 4096
_N_LAYERS = 3
_CL_RATE = 0.2
_TEMP = 0.2
_DECAY = 1e-4

_INTERPRET = False

_NC = 2
_NS = 16

_NNZ_PAD = 1638400
_EDGES_PER_TILE = _NNZ_PAD // _NS
_KB = 160
_NBLK = _EDGES_PER_TILE // _KB
_DEPTH = 4
_NPAD = 50048
_ROWS_PER_TILE = _NPAD // _NS

_mesh = plsc.VectorSubcoreMesh(core_axis_name="c", subcore_axis_name="s",
                               num_cores=_NC, num_subcores=_NS)


def _spmm_body(x_lo, x_hi, rows1, cols1, vals1, zeros, y_lo, y_hi,
               ra, rb_, rc, rd, ca, cb, cc_, cd, va, vb, vc, vd,
               xa, xb, xc, xd, acc, gsem, ssem, isem):
    c = lax.axis_index("c")
    s = lax.axis_index("s")
    bufs = ((ra, ca, va, xa), (rb_, cb, vb, xb),
            (rc, cc_, vc, xc), (rd, cd, vd, xd))

    def half(xsrc, ydst):
        pltpu.sync_copy(zeros, acc.at[pl.ds(s * _ROWS_PER_TILE, _ROWS_PER_TILE)])
        plsc.subcore_barrier()
        ebase = s * _EDGES_PER_TILE

        def fire_idx(b, bufp):
            r_, c_, v_, _ = bufp
            ee = ebase + b * _KB
            pltpu.async_copy(rows1.at[pl.ds(ee, _KB)], r_, isem)
            pltpu.async_copy(cols1.at[pl.ds(ee, _KB)], c_, isem)
            pltpu.async_copy(vals1.at[pl.ds(ee, _KB)], v_, isem)

        def drain_idx():
            pltpu.make_async_copy(rows1.at[pl.ds(0, _KB)], ra, isem).wait()
            pltpu.make_async_copy(cols1.at[pl.ds(0, _KB)], ca, isem).wait()
            pltpu.make_async_copy(vals1.at[pl.ds(0, _KB)], va, isem).wait()

        def fire_gather(bufp):
            _, c_, _, x_ = bufp
            pltpu.async_copy(xsrc.at[c_], x_, gsem)

        def drain_gather():
            pltpu.make_async_copy(xsrc.at[ca], xa, gsem).wait()

        def scale(bufp):
            _, _, v_, x_ = bufp

            def body(g, carry2):
                vv = v_[pl.ds(g * 16, 16)]
                for l in range(16):
                    vs = vv[l]
                    r = g * 16 + l
                    x_[r, pl.ds(0, 16)] = x_[r, pl.ds(0, 16)] * vs
                    x_[r, pl.ds(16, 16)] = x_[r, pl.ds(16, 16)] * vs
                return carry2

            lax.fori_loop(0, _KB // 16, body, 0)

        def fire_scatter(bufp):
            r_, _, _, x_ = bufp
            pltpu.async_copy(x_, acc.at[r_], ssem, add=True)

        def drain_scatter():
            pltpu.make_async_copy(xa, acc.at[ra], ssem).wait()

        for k in range(_DEPTH - 1):
            fire_idx(k, bufs[k])
        for k in range(_DEPTH - 1):
            drain_idx()
        fire_idx(_DEPTH - 1, bufs[_DEPTH - 1])
        for k in range(_DEPTH - 1):
            fire_gather(bufs[k])

        def outer(g2, carry):
            for p in range(_DEPTH):
                b = _DEPTH * g2 + p
                cur = bufs[p]

                @pl.when(b >= 1)
                def _():
                    drain_scatter()

                drain_gather()
                scale(cur)
                fire_scatter(cur)

                @pl.when(b < _NBLK - (_DEPTH - 1))
                def _():
                    drain_idx()
                    fire_gather(bufs[(p + _DEPTH - 1) % _DEPTH])

                @pl.when(b < _NBLK - _DEPTH)
                def _():
                    fire_idx(b + _DEPTH, cur)
            return carry

        lax.fori_loop(0, _NBLK // _DEPTH, outer, 0)
        drain_scatter()
        plsc.subcore_barrier()
        pltpu.sync_copy(acc.at[pl.ds(s * _ROWS_PER_TILE, _ROWS_PER_TILE)],
                        ydst.at[pl.ds(s * _ROWS_PER_TILE, _ROWS_PER_TILE)])

    @pl.when(c == 0)
    def _():
        half(x_lo, y_lo)

    @pl.when(c == 1)
    def _():
        half(x_hi, y_hi)


_spmm_call = pl.kernel(
    _spmm_body,
    out_type=[jax.ShapeDtypeStruct((_NPAD, _HALF), jnp.float32)] * 2,
    mesh=_mesh,
    scratch_types=(
        [pltpu.VMEM((_KB,), jnp.int32)] * 8
        + [pltpu.VMEM((_KB,), jnp.float32)] * 4
        + [pltpu.VMEM((_KB, _HALF), jnp.float32)] * 4
        + [
        pltpu.VMEM_SHARED((_NPAD, _HALF), jnp.float32),
        pltpu.SemaphoreType.DMA,
        pltpu.SemaphoreType.DMA,
        pltpu.SemaphoreType.DMA,
    ]),
    compiler_params=pltpu.CompilerParams(use_tc_tiling_on_sc=False),
)


_GROWS = _B // _NS
_GSUB = _GROWS // 128


def _gather_body(x0l, x0h, g1l, g1h, g2l, g2h, g3l, g3h,
                 a1l, a1h, a2l, a2h, a3l, a3h,
                 b1l, b1h, b2l, b2h, b3l, b3h,
                 uidx, pidx, nidx, gout,
                 ibuf, b0, b1, b2, b3, mbuf, gsem):
    c = lax.axis_index("c")
    s = lax.axis_index("s")

    def run(hh, tabs):
        x0t, g1t, g2t, g3t, a1t, a2t, a3t, b1t, b2t, b3t = tabs

        def g4(t0, t1, t2, t3):
            descs = []
            for j in range(_GSUB):
                dst = pl.ds(j * 128, 128)
                idxr = ibuf.at[s * _GSUB + j]
                descs += [
                    pltpu.async_copy(t0.at[idxr], b0.at[dst], gsem),
                    pltpu.async_copy(t1.at[idxr], b1.at[dst], gsem),
                    pltpu.async_copy(t2.at[idxr], b2.at[dst], gsem),
                    pltpu.async_copy(t3.at[idxr], b3.at[dst], gsem),
                ]
            for d in descs:
                d.wait()

        def g3only(t1, t2, t3):
            descs = []
            for j in range(_GSUB):
                dst = pl.ds(j * 128, 128)
                idxr = ibuf.at[s * _GSUB + j]
                descs += [
                    pltpu.async_copy(t1.at[idxr], b1.at[dst], gsem),
                    pltpu.async_copy(t2.at[idxr], b2.at[dst], gsem),
                    pltpu.async_copy(t3.at[idxr], b3.at[dst], gsem),
                ]
            for d in descs:
                d.wait()

        def mean4():
            def mrow(r, carry):
                for h2 in range(2):
                    sl = pl.ds(h2 * 16, 16)
                    mbuf[r, sl] = (b0[r, sl] + b1[r, sl] + b2[r, sl]
                                   + b3[r, sl]) * 0.25
                return carry
            lax.fori_loop(0, _GROWS, mrow, 0, unroll=4)

        def wout(q, src):
            pltpu.sync_copy(src, gout.at[hh, q, pl.ds(s * _GROWS, _GROWS)])

        for idx_hbm, q0, has_cl in [(uidx, 0, True), (pidx, 4, True),
                                    (nidx, 8, False)]:
            pltpu.sync_copy(idx_hbm, ibuf)
            g4(x0t, g1t, g2t, g3t)
            wout(q0, b0)
            mean4()
            wout(q0 + 1, mbuf)
            if has_cl:
                g3only(a1t, a2t, a3t)
                mean4()
                wout(q0 + 2, mbuf)
                g3only(b1t, b2t, b3t)
                mean4()
                wout(q0 + 3, mbuf)

    @pl.when(c == 0)
    def _():
        run(0, (x0l, g1l, g2l, g3l, a1l, a2l, a3l, b1l, b2l, b3l))

    @pl.when(c == 1)
    def _():
        run(1, (x0h, g1h, g2h, g3h, a1h, a2h, a3h, b1h, b2h, b3h))


_gather_call = pl.kernel(
    _gather_body,
    out_type=jax.ShapeDtypeStruct((2, 10, _B, _HALF), jnp.float32),
    mesh=_mesh,
    scratch_types=[
        pltpu.VMEM((_B // 128, 128), jnp.int32),
        pltpu.VMEM((_GROWS, _HALF), jnp.float32),
        pltpu.VMEM((_GROWS, _HALF), jnp.float32),
        pltpu.VMEM((_GROWS, _HALF), jnp.float32),
        pltpu.VMEM((_GROWS, _HALF), jnp.float32),
        pltpu.VMEM((_GROWS, _HALF), jnp.float32),
        pltpu.SemaphoreType.DMA,
    ],
    compiler_params=pltpu.CompilerParams(use_tc_tiling_on_sc=False),
)


_CH = 512
_NCH = _B // _CH


def _rownorm(x):
    ss = jnp.sum(x * x, axis=1, keepdims=True)
    nrm = jnp.sqrt(ss)
    return x / jnp.maximum(nrm, 1e-12)


def _loss_body(users_col, users_row, pos_col, pos_row,
               ue, pe, ne, ue0, pe0, ne0,
               u1, u2, i1, i2,
               mf_ref, cl_ref, reg_ref,
               mcol_s, mrow_s, n1s, n2s):
    uen = _rownorm(ue[...])
    pen = _rownorm(pe[...])
    nen = _rownorm(ne[...])
    pos_s = jnp.sum(uen * pen, axis=1, keepdims=True)
    neg_s = jnp.sum(uen * nen, axis=1, keepdims=True)
    x = pos_s - neg_s
    sig = 1.0 / (1.0 + jnp.exp(-x))
    maxi = jnp.log(sig + 1e-6)
    mf_ref[0, 0] = -jnp.sum(maxi) / _B

    reg = 0.5 * (jnp.sum(ue0[...] * ue0[...]) + jnp.sum(pe0[...] * pe0[...])
                 + jnp.sum(ne0[...] * ne0[...]))
    reg_ref[0, 0] = _DECAY * (reg / _B)

    def one_cl(ids_col, ids_row, v1, v2):
        def mask_col_body(c, carry):
            rows = ids_col[pl.ds(c * _CH, _CH), :]
            eq = rows == ids_row[...]
            ii = lax.broadcasted_iota(jnp.int32, (_CH, _B), 0) + c * _CH
            jj = lax.broadcasted_iota(jnp.int32, (_CH, _B), 1)
            dup = jnp.any(eq & (jj < ii), axis=1, keepdims=True)
            mcol_s[pl.ds(c * _CH, _CH), :] = jnp.where(dup, 0.0, 1.0)
            return carry

        lax.fori_loop(0, _NCH, mask_col_body, 0)

        def mask_row_body(c, carry):
            cols = ids_row[:, pl.ds(c * _CH, _CH)]
            eq = ids_col[...] == cols
            kk = lax.broadcasted_iota(jnp.int32, (_B, _CH), 0)
            jj = lax.broadcasted_iota(jnp.int32, (_B, _CH), 1) + c * _CH
            dup = jnp.any(eq & (kk < jj), axis=0, keepdims=True)
            mrow_s[:, pl.ds(c * _CH, _CH)] = jnp.where(dup, 0.0, 1.0)
            return carry

        lax.fori_loop(0, _NCH, mask_row_body, 0)

        count = jnp.sum(mcol_s[...])
        n1s[...] = _rownorm(v1[...])
        n2s[...] = _rownorm(v2[...])

        def chunk_body(c, acc):
            n1c = n1s[pl.ds(c * _CH, _CH), :]
            n2c = n2s[pl.ds(c * _CH, _CH), :]
            posv = jnp.exp(jnp.sum(n1c * n2c, axis=1, keepdims=True) / _TEMP)
            sm = lax.dot_general(n1c, n2s[...], (((1,), (1,)), ((), ())),
                                 preferred_element_type=jnp.float32)
            e = jnp.exp(sm / _TEMP) * mrow_s[...]
            ttl = jnp.sum(e, axis=1, keepdims=True)
            li = -jnp.log(posv / ttl + 1e-5)
            return acc + jnp.sum(li * mcol_s[pl.ds(c * _CH, _CH), :])

        tot = lax.fori_loop(0, _NCH, chunk_body, 0.0)
        return tot / count

    ucl = one_cl(users_col, users_row, u1, u2)
    icl = one_cl(pos_col, pos_row, i1, i2)
    cl_ref[0, 0] = _CL_RATE * (ucl + icl)


def _loss_call(users_col, users_row, pos_col, pos_row,
               ue, pe, ne, ue0, pe0, ne0, u1, u2, i1, i2):
    f32 = jnp.float32
    out = pl.pallas_call(
        _loss_body,
        out_shape=[jax.ShapeDtypeStruct((1, 1), f32)] * 3,
        out_specs=[pl.BlockSpec(memory_space=pltpu.SMEM)] * 3,
        scratch_shapes=[
            pltpu.VMEM((_B, 1), f32),
            pltpu.VMEM((1, _B), f32),
            pltpu.VMEM((_B, _EMB), f32),
            pltpu.VMEM((_B, _EMB), f32),
        ],
        interpret=_INTERPRET,
    )(users_col, users_row, pos_col, pos_row,
      ue, pe, ne, ue0, pe0, ne0, u1, u2, i1, i2)
    return out[0][0, 0], out[1][0, 0], out[2][0, 0]


def kernel(users, pos_items, neg_items, adj1_idx, adj1_val, adj2_idx, adj2_val,
           graph_idx, graph_val, embed_user, embed_item):
    f32 = jnp.float32
    i32 = jnp.int32

    all0 = jnp.concatenate(
        [embed_user, embed_item, jnp.zeros((_NPAD - _N, _EMB), f32)], axis=0)
    x0l = all0[:, :_HALF]
    x0h = all0[:, _HALF:]
    zeros = jnp.zeros((_ROWS_PER_TILE, _HALF), f32)

    npad = _NNZ_PAD - _NNZ

    def prep(idx, val):
        r = jnp.concatenate([idx[0].astype(i32), jnp.zeros((npad,), i32)])
        cc = jnp.concatenate([idx[1].astype(i32), jnp.zeros((npad,), i32)])
        v = jnp.concatenate([val, jnp.zeros((npad,), f32)])
        return r, cc, v

    def chain(idx, val):
        rows2, cols2, vals2 = prep(idx, val)
        outs = []
        xl, xh = x0l, x0h
        for _ in range(_N_LAYERS):
            xl, xh = _spmm_call(xl, xh, rows2, cols2, vals2, zeros)
            outs.append((xl, xh))
        return outs

    g = chain(graph_idx, graph_val)
    a = chain(adj1_idx, adj1_val)
    b = chain(adj2_idx, adj2_val)

    users_i = users.astype(i32)
    pos_i = pos_items.astype(i32)
    neg_i = neg_items.astype(i32)
    uidx = users_i.reshape(_B // 128, 128)
    pidx = (pos_i + _N_USERS).reshape(_B // 128, 128)
    nidx = (neg_i + _N_USERS).reshape(_B // 128, 128)

    G = _gather_call(x0l, x0h,
                     g[0][0], g[0][1], g[1][0], g[1][1], g[2][0], g[2][1],
                     a[0][0], a[0][1], a[1][0], a[1][1], a[2][0], a[2][1],
                     b[0][0], b[0][1], b[1][0], b[1][1], b[2][0], b[2][1],
                     uidx, pidx, nidx)

    def full(q):
        return jnp.concatenate([G[0, q], G[1, q]], axis=-1)

    ue0, ue, u1, u2 = full(0), full(1), full(2), full(3)
    pe0, pe, i1, i2 = full(4), full(5), full(6), full(7)
    ne0, ne = full(8), full(9)

    users_col = users_i.reshape(_B, 1)
    users_row = users_i.reshape(1, _B)
    pos_col = pos_i.reshape(_B, 1)
    pos_row = pos_i.reshape(1, _B)

    mf, cl, reg = _loss_call(users_col, users_row, pos_col, pos_row,
                             ue, pe, ne, ue0, pe0, ne0, u1, u2, i1, i2)
    return mf, cl, reg
